# baseline (speedup 1.0000x reference)
import functools

import jax
import jax.numpy as jnp
from jax import lax
from jax.experimental import pallas as pl
from jax.experimental.pallas import tpu as pltpu
from jax.experimental.pallas import tpu_sc as plsc

_NB = 16
_BS = 64
_HID = 128
_NACT = 16
_FEAT = 3
_ROWS_PER_TILE = 8
_NTILES = _BS // _ROWS_PER_TILE


def _dot_t(a, b):
    return lax.dot_general(a, b, (((1,), (1,)), ((), ())),
                           preferred_element_type=jnp.float32)


def _dense_kernel(x_ref, we_ref, be_ref, wg_ref, bg_ref, w1_ref, b1_ref,
                  w2_ref, b2_ref, out_ref):
    row = jax.lax.broadcasted_iota(jnp.int32, (_NB * _FEAT, _FEAT), 0)
    col = jax.lax.broadcasted_iota(jnp.int32, (_NB * _FEAT, _FEAT), 1)
    t = jnp.where(row % _FEAT == col, 1.0, 0.0)
    us = jnp.dot(x_ref[...], t, preferred_element_type=jnp.float32)
    xm = _dot_t(us, we_ref[...]) * (1.0 / _NB) + be_ref[...]
    g = _dot_t(xm, wg_ref[...]) + bg_ref[...]
    h = _dot_t(g, w1_ref[...]) + b1_ref[...]
    h = jnp.where(h >= 0, h, 0.01 * h)
    out_ref[...] = _dot_t(h, w2_ref[...]) + b2_ref[...]


_BLK = _ROWS_PER_TILE * _NACT
_ACT_OFF = _BS * _NACT


@functools.partial(
    pl.kernel,
    out_type=jax.ShapeDtypeStruct((_BS,), jnp.float32),
    mesh=plsc.VectorSubcoreMesh(core_axis_name="c", subcore_axis_name="s"),
    scratch_types=[
        pltpu.VMEM((_BLK,), jnp.float32),
        pltpu.VMEM((_BLK,), jnp.float32),
        pltpu.VMEM((_NACT,), jnp.float32),
        pltpu.SemaphoreType.DMA,
    ],
)
def _sc_select(inps_hbm, q_hbm, out_hbm, act_v, q_v, out_v, sem):
    wid = lax.axis_index("s") * 2 + lax.axis_index("c")

    @pl.when(wid < _NTILES)
    def _():
        copies = [
            pltpu.async_copy(
                inps_hbm.at[pl.ds(_ACT_OFF + wid * _BLK, _BLK)], act_v, sem),
            pltpu.async_copy(q_hbm.at[pl.ds(wid * _BLK, _BLK)], q_v, sem),
        ]
        for c in copies:
            c.wait()
        iota = lax.iota(jnp.int32, _NACT)

        def splat_reduce(v, op):
            for s in (8, 4, 2, 1):
                perm = jnp.bitwise_xor(iota, s)
                v = op(v, v.at[perm].get(mode="promise_in_bounds"))
            return v

        acc = jnp.zeros((_NACT,), jnp.float32)
        for r in range(_ROWS_PER_TILE):
            av = act_v[pl.ds(r * _NACT, _NACT)]
            m_sp = splat_reduce(av, jnp.maximum)
            idx_sp = splat_reduce(jnp.where(av == m_sp, iota, _NACT),
                                  jnp.minimum)
            q_sp = q_v[pl.ds(r * _NACT, _NACT)].at[idx_sp].get(
                mode="promise_in_bounds")
            acc = jnp.where(iota == r, q_sp, acc)
        out_v[...] = acc
        pltpu.sync_copy(out_v.at[pl.ds(0, _ROWS_PER_TILE)],
                        out_hbm.at[pl.ds(wid * _ROWS_PER_TILE,
                                         _ROWS_PER_TILE)])


def kernel(inps, unary_tensor, W_emb, b_emb, W_gcn, b_gcn, W1, b1, W2, b2):
    x = unary_tensor.reshape(_BS, _NB * _FEAT)
    all_q = pl.pallas_call(
        _dense_kernel,
        out_shape=jax.ShapeDtypeStruct((_BS, _NACT), jnp.float32),
    )(x, W_emb, b_emb.reshape(1, _HID), W_gcn, b_gcn.reshape(1, _HID),
      W1, b1.reshape(1, _HID), W2, b2.reshape(1, _NACT))
    return _sc_select(inps.reshape(-1), all_q.reshape(-1)).reshape(_BS, 1)
